# baseline (speedup 1.0000x reference)
import functools

import jax
import jax.numpy as jnp
from jax import lax
from jax.experimental import pallas as pl
from jax.experimental.pallas import tpu as pltpu
from jax.experimental.pallas import tpu_sc as plsc

N = 10000
E = 320000
NC = 2
NS = 16
NW = NC * NS
CW = 64
NROW = E // CW
ROWS_PW = 160
NROW_PAD = NW * ROWS_PW
NP = 10240
SLAB = 64
NSLAB = NP // SLAB
SLAB_PT = NSLAB // NS
K = 2
SB = 32
NSB = ROWS_PW // SB
NPAIR = SB // K
DEGW = 128


def _mesh():
    return plsc.VectorSubcoreMesh(core_axis_name="c", subcore_axis_name="s",
                                  num_cores=NC, num_subcores=NS)


def _prep_idx(v, fill):
    v2 = v.reshape(NROW, CW)
    pad = jnp.full((NROW_PAD - NROW, CW), fill, jnp.int32)
    return jnp.concatenate([v2, pad], axis=0)


def _sc_sum(g, src2d, dst2d):
    d = g.shape[1]

    @functools.partial(
        pl.kernel,
        out_type=jax.ShapeDtypeStruct((NC, NP, d), jnp.float32),
        mesh=_mesh(),
        scratch_types=[
            pltpu.VMEM((SB, CW), jnp.int32),
            pltpu.VMEM((SB, CW), jnp.int32),
            [pltpu.VMEM((CW, d), jnp.float32) for _ in range(K)],
            pltpu.VMEM_SHARED((NP, d), jnp.float32),
            pltpu.SemaphoreType.DMA,
            pltpu.SemaphoreType.DMA,
        ],
    )
    def agg(g_hbm, src_hbm, dst_hbm, zrow_hbm, sum_hbm,
            sidx, didx, rows, acc_sh, semg, sems):
        ci = lax.axis_index("c")
        si = lax.axis_index("s")
        w = si * NC + ci

        pltpu.sync_copy(zrow_hbm, rows[0])
        for k in range(SLAB_PT):
            sl = si * SLAB_PT + k
            pltpu.async_copy(rows[0], acc_sh.at[pl.ds(sl * SLAB, SLAB), :],
                             semg)
        for k in range(SLAB_PT):
            sl = si * SLAB_PT + k
            pltpu.make_async_copy(
                rows[0], acc_sh.at[pl.ds(sl * SLAB, SLAB), :], semg).wait()

        start = w * ROWS_PW
        plsc.subcore_barrier()

        def body(sb, carry):
            pltpu.sync_copy(src_hbm.at[pl.ds(start + sb * SB, SB), :], sidx)
            pltpu.sync_copy(dst_hbm.at[pl.ds(start + sb * SB, SB), :], didx)

            def pair(p, c2):
                for b in range(K):
                    i = p * K + b
                    pltpu.async_copy(g_hbm.at[sidx.at[i]], rows[b], semg)
                for b in range(K):
                    i = p * K + b
                    pltpu.make_async_copy(
                        g_hbm.at[sidx.at[i]], rows[b], semg).wait()
                    pltpu.async_copy(rows[b], acc_sh.at[didx.at[i]], sems,
                                     add=True)
                for b in range(K):
                    i = p * K + b
                    pltpu.make_async_copy(
                        rows[b], acc_sh.at[didx.at[i]], sems).wait()
                return c2

            lax.fori_loop(0, NPAIR, pair, 0)
            return carry

        lax.fori_loop(0, NSB, body, 0)
        plsc.subcore_barrier()

        for k in range(SLAB_PT):
            sl = si * SLAB_PT + k
            b = k % K
            pltpu.sync_copy(acc_sh.at[pl.ds(sl * SLAB, SLAB), :], rows[b])
            pltpu.async_copy(rows[b],
                             sum_hbm.at[ci, pl.ds(sl * SLAB, SLAB), :], sems)
        for k in range(SLAB_PT):
            sl = si * SLAB_PT + k
            b = k % K
            pltpu.make_async_copy(
                rows[b], sum_hbm.at[ci, pl.ds(sl * SLAB, SLAB), :],
                sems).wait()

    zrow = jnp.zeros((CW, d), jnp.float32)
    return agg(g, src2d, dst2d, zrow)


def _sc_deg(dst2d):
    d = DEGW

    @functools.partial(
        pl.kernel,
        out_type=jax.ShapeDtypeStruct((NC, NP, d), jnp.float32),
        mesh=_mesh(),
        scratch_types=[
            pltpu.VMEM((SB, CW), jnp.int32),
            pltpu.VMEM((CW, d), jnp.float32),
            pltpu.VMEM((SLAB, d), jnp.float32),
            pltpu.VMEM_SHARED((NP, d), jnp.float32),
            pltpu.SemaphoreType.DMA,
            pltpu.SemaphoreType.DMA,
        ],
    )
    def deg(dst_hbm, zrow_hbm, ones_hbm, deg_hbm,
            didx, ones_v, zbuf, deg_sh, semg, sems):
        ci = lax.axis_index("c")
        si = lax.axis_index("s")
        w = si * NC + ci

        pltpu.sync_copy(zrow_hbm, zbuf)
        for k in range(SLAB_PT):
            sl = si * SLAB_PT + k
            pltpu.async_copy(zbuf, deg_sh.at[pl.ds(sl * SLAB, SLAB), :], semg)
        for k in range(SLAB_PT):
            sl = si * SLAB_PT + k
            pltpu.make_async_copy(
                zbuf, deg_sh.at[pl.ds(sl * SLAB, SLAB), :], semg).wait()

        pltpu.sync_copy(ones_hbm, ones_v)
        start = w * ROWS_PW
        plsc.subcore_barrier()

        def body(sb, carry):
            pltpu.sync_copy(dst_hbm.at[pl.ds(start + sb * SB, SB), :], didx)

            def pair(p, c2):
                for b in range(K):
                    i = p * K + b
                    pltpu.async_copy(ones_v, deg_sh.at[didx.at[i]], sems,
                                     add=True)
                for b in range(K):
                    i = p * K + b
                    pltpu.make_async_copy(
                        ones_v, deg_sh.at[didx.at[i]], sems).wait()
                return c2

            lax.fori_loop(0, NPAIR, pair, 0)
            return carry

        lax.fori_loop(0, NSB, body, 0)
        plsc.subcore_barrier()

        for k in range(SLAB_PT):
            sl = si * SLAB_PT + k
            pltpu.sync_copy(deg_sh.at[pl.ds(sl * SLAB, SLAB), :], zbuf)
            pltpu.async_copy(zbuf,
                             deg_hbm.at[ci, pl.ds(sl * SLAB, SLAB), :], sems)
            pltpu.make_async_copy(
                zbuf, deg_hbm.at[ci, pl.ds(sl * SLAB, SLAB), :], sems).wait()

    zrow = jnp.zeros((SLAB, d), jnp.float32)
    ones = jnp.ones((CW, d), jnp.float32)
    return deg(dst2d, zrow, ones)


_BN = 2000


def _tc_first(h, w_neigh, w_self, b):
    d_out = w_neigh.shape[1]

    def body(h_ref, wn_ref, ws_ref, b_ref, g_ref, s_ref):
        h_blk = h_ref[...]
        g_ref[...] = jnp.dot(h_blk, wn_ref[...],
                             preferred_element_type=jnp.float32)
        s_ref[...] = jnp.dot(h_blk, ws_ref[...],
                             preferred_element_type=jnp.float32) + b_ref[...]

    return pl.pallas_call(
        body,
        grid=(N // _BN,),
        in_specs=[
            pl.BlockSpec((_BN, h.shape[1]), lambda i: (i, 0)),
            pl.BlockSpec(w_neigh.shape, lambda i: (0, 0)),
            pl.BlockSpec(w_self.shape, lambda i: (0, 0)),
            pl.BlockSpec((1, d_out), lambda i: (0, 0)),
        ],
        out_specs=[
            pl.BlockSpec((_BN, d_out), lambda i: (i, 0)),
            pl.BlockSpec((_BN, d_out), lambda i: (i, 0)),
        ],
        out_shape=[
            jax.ShapeDtypeStruct((N, d_out), jnp.float32),
            jax.ShapeDtypeStruct((N, d_out), jnp.float32),
        ],
    )(h, w_neigh, w_self, b.reshape(1, -1))


def _tc_mid(s_prev, ssum, deg, w_neigh, w_self, b):
    d_in = s_prev.shape[1]
    d_out = w_neigh.shape[1]

    def body(sp_ref, sum_ref, deg_ref, wn_ref, ws_ref, b_ref, g_ref, s_ref):
        total = sum_ref[0] + sum_ref[1]
        degc = deg_ref[0, :, 0:1] + deg_ref[1, :, 0:1]
        h_blk = jnp.maximum(
            sp_ref[...] + total / jnp.maximum(degc, 1.0), 0.0)
        g_ref[...] = jnp.dot(h_blk, wn_ref[...],
                             preferred_element_type=jnp.float32)
        s_ref[...] = jnp.dot(h_blk, ws_ref[...],
                             preferred_element_type=jnp.float32) + b_ref[...]

    return pl.pallas_call(
        body,
        grid=(N // _BN,),
        in_specs=[
            pl.BlockSpec((_BN, d_in), lambda i: (i, 0)),
            pl.BlockSpec((NC, _BN, d_in), lambda i: (0, i, 0)),
            pl.BlockSpec((NC, _BN, DEGW), lambda i: (0, i, 0)),
            pl.BlockSpec(w_neigh.shape, lambda i: (0, 0)),
            pl.BlockSpec(w_self.shape, lambda i: (0, 0)),
            pl.BlockSpec((1, d_out), lambda i: (0, 0)),
        ],
        out_specs=[
            pl.BlockSpec((_BN, d_out), lambda i: (i, 0)),
            pl.BlockSpec((_BN, d_out), lambda i: (i, 0)),
        ],
        out_shape=[
            jax.ShapeDtypeStruct((N, d_out), jnp.float32),
            jax.ShapeDtypeStruct((N, d_out), jnp.float32),
        ],
    )(s_prev, ssum, deg, w_neigh, w_self, b.reshape(1, -1))


def _tc_mid2(s_prev, ssum, deg, w_self, b):
    d_in = s_prev.shape[1]
    d_out = w_self.shape[1]

    def body(sp_ref, sum_ref, deg_ref, ws_ref, b_ref, h_ref, s_ref):
        total = sum_ref[0] + sum_ref[1]
        degc = deg_ref[0, :, 0:1] + deg_ref[1, :, 0:1]
        h_blk = jnp.maximum(
            sp_ref[...] + total / jnp.maximum(degc, 1.0), 0.0)
        h_ref[...] = h_blk
        s_ref[...] = jnp.dot(h_blk, ws_ref[...],
                             preferred_element_type=jnp.float32) + b_ref[...]

    return pl.pallas_call(
        body,
        grid=(N // _BN,),
        in_specs=[
            pl.BlockSpec((_BN, d_in), lambda i: (i, 0)),
            pl.BlockSpec((NC, _BN, d_in), lambda i: (0, i, 0)),
            pl.BlockSpec((NC, _BN, DEGW), lambda i: (0, i, 0)),
            pl.BlockSpec(w_self.shape, lambda i: (0, 0)),
            pl.BlockSpec((1, d_out), lambda i: (0, 0)),
        ],
        out_specs=[
            pl.BlockSpec((_BN, d_in), lambda i: (i, 0)),
            pl.BlockSpec((_BN, d_out), lambda i: (i, 0)),
        ],
        out_shape=[
            jax.ShapeDtypeStruct((N, d_in), jnp.float32),
            jax.ShapeDtypeStruct((N, d_out), jnp.float32),
        ],
    )(s_prev, ssum, deg, w_self, b.reshape(1, -1))


def _tc_last(s_prev, ssum, deg, w_neigh):
    d_in = w_neigh.shape[0]
    d_out = w_neigh.shape[1]

    def body(sp_ref, sum_ref, deg_ref, wn_ref, o_ref):
        total = sum_ref[0] + sum_ref[1]
        degc = deg_ref[0, :, 0:1] + deg_ref[1, :, 0:1]
        h_neigh = total / jnp.maximum(degc, 1.0)
        o_ref[...] = sp_ref[...] + jnp.dot(
            h_neigh, wn_ref[...], preferred_element_type=jnp.float32)

    return pl.pallas_call(
        body,
        grid=(N // _BN,),
        in_specs=[
            pl.BlockSpec((_BN, d_out), lambda i: (i, 0)),
            pl.BlockSpec((NC, _BN, d_in), lambda i: (0, i, 0)),
            pl.BlockSpec((NC, _BN, DEGW), lambda i: (0, i, 0)),
            pl.BlockSpec(w_neigh.shape, lambda i: (0, 0)),
        ],
        out_specs=pl.BlockSpec((_BN, d_out), lambda i: (i, 0)),
        out_shape=jax.ShapeDtypeStruct((N, d_out), jnp.float32),
    )(s_prev, ssum, deg, w_neigh)


def kernel(inputs, edge_index0, edge_index1, edge_index2,
           W_self0, W_neigh0, b0, W_self1, W_neigh1, b1,
           W_self2, W_neigh2, b2):
    s0_2d = _prep_idx(edge_index0[0], 0)
    d0_2d = _prep_idx(edge_index0[1], N)
    s1_2d = _prep_idx(edge_index1[0], 0)
    d1_2d = _prep_idx(edge_index1[1], N)
    s2_2d = _prep_idx(edge_index2[0], 0)
    d2_2d = _prep_idx(edge_index2[1], N)

    g0, s0 = _tc_first(inputs, W_neigh0, W_self0, b0)
    sum0 = _sc_sum(g0, s0_2d, d0_2d)
    deg0 = _sc_deg(d0_2d)
    g1, s1 = _tc_mid(s0, sum0, deg0, W_neigh1, W_self1, b1)
    sum1 = _sc_sum(g1, s1_2d, d1_2d)
    deg1 = _sc_deg(d1_2d)
    h2, s2 = _tc_mid2(s1, sum1, deg1, W_self2, b2)
    sum2 = _sc_sum(h2, s2_2d, d2_2d)
    deg2 = _sc_deg(d2_2d)
    return _tc_last(s2, sum2, deg2, W_neigh2)


# speedup vs baseline: 1.8108x; 1.8108x over previous
import functools

import jax
import jax.numpy as jnp
from jax import lax
from jax.experimental import pallas as pl
from jax.experimental.pallas import tpu as pltpu
from jax.experimental.pallas import tpu_sc as plsc

N = 10000
E = 320000
NC = 2
NS = 16
NW = NC * NS
EPW = E // NW
C = 80
NCHUNK = EPW // C
NPAIR = NCHUNK // 2
RPT = 640
DEGW = 128


def _mesh():
    return plsc.VectorSubcoreMesh(core_axis_name="c", subcore_axis_name="s",
                                  num_cores=NC, num_subcores=NS)


def _row_chunks(si):
    return jnp.where(si == NS - 1, (N - (NS - 1) * RPT) // C, RPT // C)


def _sc_sum(g, src, dst):
    d = g.shape[1]

    @functools.partial(
        pl.kernel,
        out_type=jax.ShapeDtypeStruct((NC, N, d), jnp.float32),
        mesh=_mesh(),
        scratch_types=[
            [pltpu.VMEM((C,), jnp.int32) for _ in range(2)],
            [pltpu.VMEM((C,), jnp.int32) for _ in range(2)],
            [pltpu.VMEM((C, d), jnp.float32) for _ in range(2)],
            pltpu.VMEM_SHARED((N, d), jnp.float32),
            [pltpu.SemaphoreType.DMA for _ in range(2)],
            [pltpu.SemaphoreType.DMA for _ in range(2)],
            [pltpu.SemaphoreType.DMA for _ in range(2)],
        ],
    )
    def agg(g_hbm, src_hbm, dst_hbm, zrow_hbm, sum_hbm,
            sidx, didx, rows, acc_sh, semi, semg, sems):
        ci = lax.axis_index("c")
        si = lax.axis_index("s")
        rbase = si * RPT
        nrch = _row_chunks(si)

        pltpu.sync_copy(zrow_hbm, rows[0])

        def zbody(j, carry):
            pltpu.sync_copy(rows[0], acc_sh.at[pl.ds(rbase + j * C, C), :])
            return carry

        lax.fori_loop(0, nrch, zbody, 0)
        plsc.subcore_barrier()

        ebase = (si * NC + ci) * EPW

        def body(j, carry):
            off0 = ebase + (2 * j) * C
            off1 = off0 + C
            pltpu.async_copy(src_hbm.at[pl.ds(off0, C)], sidx[0], semi[0])
            pltpu.async_copy(dst_hbm.at[pl.ds(off0, C)], didx[0], semi[0])
            pltpu.async_copy(src_hbm.at[pl.ds(off1, C)], sidx[1], semi[1])
            pltpu.async_copy(dst_hbm.at[pl.ds(off1, C)], didx[1], semi[1])
            pltpu.make_async_copy(src_hbm.at[pl.ds(off0, C)], sidx[0],
                                  semi[0]).wait()
            pltpu.make_async_copy(dst_hbm.at[pl.ds(off0, C)], didx[0],
                                  semi[0]).wait()
            pltpu.async_copy(g_hbm.at[sidx[0]], rows[0], semg[0])
            pltpu.make_async_copy(src_hbm.at[pl.ds(off1, C)], sidx[1],
                                  semi[1]).wait()
            pltpu.make_async_copy(dst_hbm.at[pl.ds(off1, C)], didx[1],
                                  semi[1]).wait()
            pltpu.async_copy(g_hbm.at[sidx[1]], rows[1], semg[1])
            pltpu.make_async_copy(g_hbm.at[sidx[0]], rows[0], semg[0]).wait()
            pltpu.async_copy(rows[0], acc_sh.at[didx[0]], sems[0], add=True)
            pltpu.make_async_copy(g_hbm.at[sidx[1]], rows[1], semg[1]).wait()
            pltpu.async_copy(rows[1], acc_sh.at[didx[1]], sems[1], add=True)
            pltpu.make_async_copy(rows[0], acc_sh.at[didx[0]],
                                  sems[0]).wait()
            pltpu.make_async_copy(rows[1], acc_sh.at[didx[1]],
                                  sems[1]).wait()
            return carry

        lax.fori_loop(0, NPAIR, body, 0)

        offt = ebase + (NCHUNK - 1) * C
        pltpu.sync_copy(src_hbm.at[pl.ds(offt, C)], sidx[0])
        pltpu.sync_copy(dst_hbm.at[pl.ds(offt, C)], didx[0])
        pltpu.async_copy(g_hbm.at[sidx[0]], rows[0], semg[0]).wait()
        pltpu.sync_copy(rows[0], acc_sh.at[didx[0]], add=True)
        plsc.subcore_barrier()

        def pbody(j, carry):
            off = rbase + j * C
            pltpu.sync_copy(acc_sh.at[pl.ds(off, C), :], rows[0])
            pltpu.sync_copy(rows[0], sum_hbm.at[ci, pl.ds(off, C), :])
            return carry

        lax.fori_loop(0, nrch, pbody, 0)

    zrow = jnp.zeros((C, d), jnp.float32)
    return agg(g, src, dst, zrow)


def _sc_deg(dst):
    d = DEGW

    @functools.partial(
        pl.kernel,
        out_type=jax.ShapeDtypeStruct((NC, N, d), jnp.float32),
        mesh=_mesh(),
        scratch_types=[
            [pltpu.VMEM((C,), jnp.int32) for _ in range(2)],
            pltpu.VMEM((C, d), jnp.float32),
            pltpu.VMEM_SHARED((N, d), jnp.float32),
            [pltpu.SemaphoreType.DMA for _ in range(2)],
            [pltpu.SemaphoreType.DMA for _ in range(2)],
        ],
    )
    def deg(dst_hbm, zrow_hbm, ones_hbm, deg_hbm,
            didx, buf, deg_sh, semi, sems):
        ci = lax.axis_index("c")
        si = lax.axis_index("s")
        rbase = si * RPT
        nrch = _row_chunks(si)

        pltpu.sync_copy(zrow_hbm, buf)

        def zbody(j, carry):
            pltpu.sync_copy(buf, deg_sh.at[pl.ds(rbase + j * C, C), :])
            return carry

        lax.fori_loop(0, nrch, zbody, 0)
        pltpu.sync_copy(ones_hbm, buf)
        plsc.subcore_barrier()

        ebase = (si * NC + ci) * EPW

        def body(j, carry):
            off0 = ebase + (2 * j) * C
            off1 = off0 + C
            pltpu.async_copy(dst_hbm.at[pl.ds(off0, C)], didx[0], semi[0])
            pltpu.async_copy(dst_hbm.at[pl.ds(off1, C)], didx[1], semi[1])
            pltpu.make_async_copy(dst_hbm.at[pl.ds(off0, C)], didx[0],
                                  semi[0]).wait()
            pltpu.async_copy(buf, deg_sh.at[didx[0]], sems[0], add=True)
            pltpu.make_async_copy(dst_hbm.at[pl.ds(off1, C)], didx[1],
                                  semi[1]).wait()
            pltpu.async_copy(buf, deg_sh.at[didx[1]], sems[1], add=True)
            pltpu.make_async_copy(buf, deg_sh.at[didx[0]], sems[0]).wait()
            pltpu.make_async_copy(buf, deg_sh.at[didx[1]], sems[1]).wait()
            return carry

        lax.fori_loop(0, NPAIR, body, 0)
        offt = ebase + (NCHUNK - 1) * C
        pltpu.sync_copy(dst_hbm.at[pl.ds(offt, C)], didx[0])
        pltpu.sync_copy(buf, deg_sh.at[didx[0]], add=True)
        plsc.subcore_barrier()

        def pbody(j, carry):
            off = rbase + j * C
            pltpu.sync_copy(deg_sh.at[pl.ds(off, C), :], buf)
            pltpu.sync_copy(buf, deg_hbm.at[ci, pl.ds(off, C), :])
            return carry

        lax.fori_loop(0, nrch, pbody, 0)

    zrow = jnp.zeros((C, d), jnp.float32)
    ones = jnp.ones((C, d), jnp.float32)
    return deg(dst, zrow, ones)


_BN = 2000


def _tc_first(h, w_neigh, w_self, b):
    d_out = w_neigh.shape[1]

    def body(h_ref, wn_ref, ws_ref, b_ref, g_ref, s_ref):
        h_blk = h_ref[...]
        g_ref[...] = jnp.dot(h_blk, wn_ref[...],
                             preferred_element_type=jnp.float32)
        s_ref[...] = jnp.dot(h_blk, ws_ref[...],
                             preferred_element_type=jnp.float32) + b_ref[...]

    return pl.pallas_call(
        body,
        grid=(N // _BN,),
        in_specs=[
            pl.BlockSpec((_BN, h.shape[1]), lambda i: (i, 0)),
            pl.BlockSpec(w_neigh.shape, lambda i: (0, 0)),
            pl.BlockSpec(w_self.shape, lambda i: (0, 0)),
            pl.BlockSpec((1, d_out), lambda i: (0, 0)),
        ],
        out_specs=[
            pl.BlockSpec((_BN, d_out), lambda i: (i, 0)),
            pl.BlockSpec((_BN, d_out), lambda i: (i, 0)),
        ],
        out_shape=[
            jax.ShapeDtypeStruct((N, d_out), jnp.float32),
            jax.ShapeDtypeStruct((N, d_out), jnp.float32),
        ],
    )(h, w_neigh, w_self, b.reshape(1, -1))


def _tc_mid(s_prev, ssum, deg, w_neigh, w_self, b):
    d_in = s_prev.shape[1]
    d_out = w_neigh.shape[1]

    def body(sp_ref, sum_ref, deg_ref, wn_ref, ws_ref, b_ref, g_ref, s_ref):
        total = sum_ref[0] + sum_ref[1]
        degc = deg_ref[0, :, 0:1] + deg_ref[1, :, 0:1]
        h_blk = jnp.maximum(
            sp_ref[...] + total / jnp.maximum(degc, 1.0), 0.0)
        g_ref[...] = jnp.dot(h_blk, wn_ref[...],
                             preferred_element_type=jnp.float32)
        s_ref[...] = jnp.dot(h_blk, ws_ref[...],
                             preferred_element_type=jnp.float32) + b_ref[...]

    return pl.pallas_call(
        body,
        grid=(N // _BN,),
        in_specs=[
            pl.BlockSpec((_BN, d_in), lambda i: (i, 0)),
            pl.BlockSpec((NC, _BN, d_in), lambda i: (0, i, 0)),
            pl.BlockSpec((NC, _BN, DEGW), lambda i: (0, i, 0)),
            pl.BlockSpec(w_neigh.shape, lambda i: (0, 0)),
            pl.BlockSpec(w_self.shape, lambda i: (0, 0)),
            pl.BlockSpec((1, d_out), lambda i: (0, 0)),
        ],
        out_specs=[
            pl.BlockSpec((_BN, d_out), lambda i: (i, 0)),
            pl.BlockSpec((_BN, d_out), lambda i: (i, 0)),
        ],
        out_shape=[
            jax.ShapeDtypeStruct((N, d_out), jnp.float32),
            jax.ShapeDtypeStruct((N, d_out), jnp.float32),
        ],
    )(s_prev, ssum, deg, w_neigh, w_self, b.reshape(1, -1))


def _tc_mid2(s_prev, ssum, deg, w_self, b):
    d_in = s_prev.shape[1]
    d_out = w_self.shape[1]

    def body(sp_ref, sum_ref, deg_ref, ws_ref, b_ref, h_ref, s_ref):
        total = sum_ref[0] + sum_ref[1]
        degc = deg_ref[0, :, 0:1] + deg_ref[1, :, 0:1]
        h_blk = jnp.maximum(
            sp_ref[...] + total / jnp.maximum(degc, 1.0), 0.0)
        h_ref[...] = h_blk
        s_ref[...] = jnp.dot(h_blk, ws_ref[...],
                             preferred_element_type=jnp.float32) + b_ref[...]

    return pl.pallas_call(
        body,
        grid=(N // _BN,),
        in_specs=[
            pl.BlockSpec((_BN, d_in), lambda i: (i, 0)),
            pl.BlockSpec((NC, _BN, d_in), lambda i: (0, i, 0)),
            pl.BlockSpec((NC, _BN, DEGW), lambda i: (0, i, 0)),
            pl.BlockSpec(w_self.shape, lambda i: (0, 0)),
            pl.BlockSpec((1, d_out), lambda i: (0, 0)),
        ],
        out_specs=[
            pl.BlockSpec((_BN, d_in), lambda i: (i, 0)),
            pl.BlockSpec((_BN, d_out), lambda i: (i, 0)),
        ],
        out_shape=[
            jax.ShapeDtypeStruct((N, d_in), jnp.float32),
            jax.ShapeDtypeStruct((N, d_out), jnp.float32),
        ],
    )(s_prev, ssum, deg, w_self, b.reshape(1, -1))


def _tc_last(s_prev, ssum, deg, w_neigh):
    d_in = w_neigh.shape[0]
    d_out = w_neigh.shape[1]

    def body(sp_ref, sum_ref, deg_ref, wn_ref, o_ref):
        total = sum_ref[0] + sum_ref[1]
        degc = deg_ref[0, :, 0:1] + deg_ref[1, :, 0:1]
        h_neigh = total / jnp.maximum(degc, 1.0)
        o_ref[...] = sp_ref[...] + jnp.dot(
            h_neigh, wn_ref[...], preferred_element_type=jnp.float32)

    return pl.pallas_call(
        body,
        grid=(N // _BN,),
        in_specs=[
            pl.BlockSpec((_BN, d_out), lambda i: (i, 0)),
            pl.BlockSpec((NC, _BN, d_in), lambda i: (0, i, 0)),
            pl.BlockSpec((NC, _BN, DEGW), lambda i: (0, i, 0)),
            pl.BlockSpec(w_neigh.shape, lambda i: (0, 0)),
        ],
        out_specs=pl.BlockSpec((_BN, d_out), lambda i: (i, 0)),
        out_shape=jax.ShapeDtypeStruct((N, d_out), jnp.float32),
    )(s_prev, ssum, deg, w_neigh)


def kernel(inputs, edge_index0, edge_index1, edge_index2,
           W_self0, W_neigh0, b0, W_self1, W_neigh1, b1,
           W_self2, W_neigh2, b2):
    g0, s0 = _tc_first(inputs, W_neigh0, W_self0, b0)
    sum0 = _sc_sum(g0, edge_index0[0], edge_index0[1])
    deg0 = _sc_deg(edge_index0[1])
    g1, s1 = _tc_mid(s0, sum0, deg0, W_neigh1, W_self1, b1)
    sum1 = _sc_sum(g1, edge_index1[0], edge_index1[1])
    deg1 = _sc_deg(edge_index1[1])
    h2, s2 = _tc_mid2(s1, sum1, deg1, W_self2, b2)
    sum2 = _sc_sum(h2, edge_index2[0], edge_index2[1])
    deg2 = _sc_deg(edge_index2[1])
    return _tc_last(s2, sum2, deg2, W_neigh2)


# speedup vs baseline: 1.8965x; 1.0473x over previous
import functools

import jax
import jax.numpy as jnp
from jax import lax
from jax.experimental import pallas as pl
from jax.experimental.pallas import tpu as pltpu
from jax.experimental.pallas import tpu_sc as plsc

N = 10000
E = 320000
NC = 2
NS = 16
NW = NC * NS
EPW = E // NW
C = 80
NCHUNK = EPW // C
NPAIR = NCHUNK // 2
RPT = 640
DEGW = 128


def _mesh():
    return plsc.VectorSubcoreMesh(core_axis_name="c", subcore_axis_name="s",
                                  num_cores=NC, num_subcores=NS)


def _row_chunks(si):
    return jnp.where(si == NS - 1, (N - (NS - 1) * RPT) // C, RPT // C)


def _sc_sum(g, src, dst):
    d = g.shape[1]

    @functools.partial(
        pl.kernel,
        out_type=jax.ShapeDtypeStruct((NC, N, d), jnp.float32),
        mesh=_mesh(),
        scratch_types=[
            [pltpu.VMEM((C,), jnp.int32) for _ in range(2)],
            [pltpu.VMEM((C,), jnp.int32) for _ in range(2)],
            [pltpu.VMEM((C, d), jnp.float32) for _ in range(2)],
            pltpu.VMEM_SHARED((N, d), jnp.float32),
            [pltpu.SemaphoreType.DMA for _ in range(2)],
            [pltpu.SemaphoreType.DMA for _ in range(2)],
            [pltpu.SemaphoreType.DMA for _ in range(2)],
        ],
    )
    def agg(g_hbm, src_hbm, dst_hbm, zrow_hbm, sum_hbm,
            sidx, didx, rows, acc_sh, semi, semg, sems):
        ci = lax.axis_index("c")
        si = lax.axis_index("s")
        rbase = si * RPT
        nrch = _row_chunks(si)

        pltpu.sync_copy(zrow_hbm, rows[0])

        def zbody(j, carry):
            pltpu.sync_copy(rows[0], acc_sh.at[pl.ds(rbase + j * C, C), :])
            return carry

        lax.fori_loop(0, nrch, zbody, 0)
        plsc.subcore_barrier()

        ebase = (si * NC + ci) * EPW

        def body(j, carry):
            off0 = ebase + (2 * j) * C
            off1 = off0 + C
            pltpu.async_copy(src_hbm.at[pl.ds(off0, C)], sidx[0], semi[0])
            pltpu.async_copy(dst_hbm.at[pl.ds(off0, C)], didx[0], semi[0])
            pltpu.async_copy(src_hbm.at[pl.ds(off1, C)], sidx[1], semi[1])
            pltpu.async_copy(dst_hbm.at[pl.ds(off1, C)], didx[1], semi[1])
            pltpu.make_async_copy(src_hbm.at[pl.ds(off0, C)], sidx[0],
                                  semi[0]).wait()
            pltpu.make_async_copy(dst_hbm.at[pl.ds(off0, C)], didx[0],
                                  semi[0]).wait()
            pltpu.async_copy(g_hbm.at[sidx[0]], rows[0], semg[0])
            pltpu.make_async_copy(src_hbm.at[pl.ds(off1, C)], sidx[1],
                                  semi[1]).wait()
            pltpu.make_async_copy(dst_hbm.at[pl.ds(off1, C)], didx[1],
                                  semi[1]).wait()
            pltpu.async_copy(g_hbm.at[sidx[1]], rows[1], semg[1])
            pltpu.make_async_copy(g_hbm.at[sidx[0]], rows[0], semg[0]).wait()
            pltpu.async_copy(rows[0], acc_sh.at[didx[0]], sems[0], add=True)
            pltpu.make_async_copy(g_hbm.at[sidx[1]], rows[1], semg[1]).wait()
            pltpu.async_copy(rows[1], acc_sh.at[didx[1]], sems[1], add=True)
            pltpu.make_async_copy(rows[0], acc_sh.at[didx[0]],
                                  sems[0]).wait()
            pltpu.make_async_copy(rows[1], acc_sh.at[didx[1]],
                                  sems[1]).wait()
            return carry

        lax.fori_loop(0, NPAIR, body, 0)

        offt = ebase + (NCHUNK - 1) * C
        pltpu.sync_copy(src_hbm.at[pl.ds(offt, C)], sidx[0])
        pltpu.sync_copy(dst_hbm.at[pl.ds(offt, C)], didx[0])
        pltpu.async_copy(g_hbm.at[sidx[0]], rows[0], semg[0]).wait()
        pltpu.sync_copy(rows[0], acc_sh.at[didx[0]], add=True)
        plsc.subcore_barrier()

        def pbody(j, carry):
            off = rbase + j * C
            pltpu.sync_copy(acc_sh.at[pl.ds(off, C), :], rows[0])
            pltpu.sync_copy(rows[0], sum_hbm.at[ci, pl.ds(off, C), :])
            return carry

        lax.fori_loop(0, nrch, pbody, 0)

    zrow = jnp.zeros((C, d), jnp.float32)
    return agg(g, src, dst, zrow)


def _sc_deg(dst):
    d = DEGW

    @functools.partial(
        pl.kernel,
        out_type=jax.ShapeDtypeStruct((NC, N, d), jnp.float32),
        mesh=_mesh(),
        scratch_types=[
            [pltpu.VMEM((C,), jnp.int32) for _ in range(4)],
            pltpu.VMEM((C, d), jnp.float32),
            pltpu.VMEM_SHARED((N, d), jnp.float32),
            [pltpu.SemaphoreType.DMA for _ in range(4)],
            [pltpu.SemaphoreType.DMA for _ in range(4)],
        ],
    )
    def deg(dst_hbm, zrow_hbm, ones_hbm, deg_hbm,
            didx, buf, deg_sh, semi, sems):
        ci = lax.axis_index("c")
        si = lax.axis_index("s")
        rbase = si * RPT
        nrch = _row_chunks(si)

        pltpu.sync_copy(zrow_hbm, buf)

        def zbody(j, carry):
            pltpu.sync_copy(buf, deg_sh.at[pl.ds(rbase + j * C, C), :])
            return carry

        lax.fori_loop(0, nrch, zbody, 0)
        pltpu.sync_copy(ones_hbm, buf)
        plsc.subcore_barrier()

        ebase = (si * NC + ci) * EPW
        nquad = NCHUNK // 4

        def body(j, carry):
            offs = [ebase + (4 * j + b) * C for b in range(4)]
            for b in range(4):
                pltpu.async_copy(dst_hbm.at[pl.ds(offs[b], C)], didx[b],
                                 semi[b])
            for b in range(4):
                pltpu.make_async_copy(dst_hbm.at[pl.ds(offs[b], C)], didx[b],
                                      semi[b]).wait()
                pltpu.async_copy(buf, deg_sh.at[didx[b]], sems[b], add=True)
            for b in range(4):
                pltpu.make_async_copy(buf, deg_sh.at[didx[b]],
                                      sems[b]).wait()
            return carry

        lax.fori_loop(0, nquad, body, 0)
        offt = ebase + (NCHUNK - 1) * C
        pltpu.sync_copy(dst_hbm.at[pl.ds(offt, C)], didx[0])
        pltpu.sync_copy(buf, deg_sh.at[didx[0]], add=True)
        plsc.subcore_barrier()

        def pbody(j, carry):
            off = rbase + j * C
            pltpu.sync_copy(deg_sh.at[pl.ds(off, C), :], buf)
            pltpu.sync_copy(buf, deg_hbm.at[ci, pl.ds(off, C), :])
            return carry

        lax.fori_loop(0, nrch, pbody, 0)

    zrow = jnp.zeros((C, d), jnp.float32)
    ones = jnp.ones((C, d), jnp.float32)
    return deg(dst, zrow, ones)


_BN = 2000


def _tc_first(h, w_neigh, w_self, b):
    d_out = w_neigh.shape[1]

    def body(h_ref, wn_ref, ws_ref, b_ref, g_ref, s_ref):
        h_blk = h_ref[...]
        g_ref[...] = jnp.dot(h_blk, wn_ref[...],
                             preferred_element_type=jnp.float32)
        s_ref[...] = jnp.dot(h_blk, ws_ref[...],
                             preferred_element_type=jnp.float32) + b_ref[...]

    return pl.pallas_call(
        body,
        grid=(N // _BN,),
        in_specs=[
            pl.BlockSpec((_BN, h.shape[1]), lambda i: (i, 0)),
            pl.BlockSpec(w_neigh.shape, lambda i: (0, 0)),
            pl.BlockSpec(w_self.shape, lambda i: (0, 0)),
            pl.BlockSpec((1, d_out), lambda i: (0, 0)),
        ],
        out_specs=[
            pl.BlockSpec((_BN, d_out), lambda i: (i, 0)),
            pl.BlockSpec((_BN, d_out), lambda i: (i, 0)),
        ],
        out_shape=[
            jax.ShapeDtypeStruct((N, d_out), jnp.float32),
            jax.ShapeDtypeStruct((N, d_out), jnp.float32),
        ],
    )(h, w_neigh, w_self, b.reshape(1, -1))


def _tc_mid(s_prev, ssum, deg, w_neigh, w_self, b):
    d_in = s_prev.shape[1]
    d_out = w_neigh.shape[1]

    def body(sp_ref, sum_ref, deg_ref, wn_ref, ws_ref, b_ref, g_ref, s_ref):
        total = sum_ref[0] + sum_ref[1]
        degc = deg_ref[0, :, 0:1] + deg_ref[1, :, 0:1]
        h_blk = jnp.maximum(
            sp_ref[...] + total / jnp.maximum(degc, 1.0), 0.0)
        g_ref[...] = jnp.dot(h_blk, wn_ref[...],
                             preferred_element_type=jnp.float32)
        s_ref[...] = jnp.dot(h_blk, ws_ref[...],
                             preferred_element_type=jnp.float32) + b_ref[...]

    return pl.pallas_call(
        body,
        grid=(N // _BN,),
        in_specs=[
            pl.BlockSpec((_BN, d_in), lambda i: (i, 0)),
            pl.BlockSpec((NC, _BN, d_in), lambda i: (0, i, 0)),
            pl.BlockSpec((NC, _BN, DEGW), lambda i: (0, i, 0)),
            pl.BlockSpec(w_neigh.shape, lambda i: (0, 0)),
            pl.BlockSpec(w_self.shape, lambda i: (0, 0)),
            pl.BlockSpec((1, d_out), lambda i: (0, 0)),
        ],
        out_specs=[
            pl.BlockSpec((_BN, d_out), lambda i: (i, 0)),
            pl.BlockSpec((_BN, d_out), lambda i: (i, 0)),
        ],
        out_shape=[
            jax.ShapeDtypeStruct((N, d_out), jnp.float32),
            jax.ShapeDtypeStruct((N, d_out), jnp.float32),
        ],
    )(s_prev, ssum, deg, w_neigh, w_self, b.reshape(1, -1))


def _tc_mid2(s_prev, ssum, deg, w_self, b):
    d_in = s_prev.shape[1]
    d_out = w_self.shape[1]

    def body(sp_ref, sum_ref, deg_ref, ws_ref, b_ref, h_ref, s_ref):
        total = sum_ref[0] + sum_ref[1]
        degc = deg_ref[0, :, 0:1] + deg_ref[1, :, 0:1]
        h_blk = jnp.maximum(
            sp_ref[...] + total / jnp.maximum(degc, 1.0), 0.0)
        h_ref[...] = h_blk
        s_ref[...] = jnp.dot(h_blk, ws_ref[...],
                             preferred_element_type=jnp.float32) + b_ref[...]

    return pl.pallas_call(
        body,
        grid=(N // _BN,),
        in_specs=[
            pl.BlockSpec((_BN, d_in), lambda i: (i, 0)),
            pl.BlockSpec((NC, _BN, d_in), lambda i: (0, i, 0)),
            pl.BlockSpec((NC, _BN, DEGW), lambda i: (0, i, 0)),
            pl.BlockSpec(w_self.shape, lambda i: (0, 0)),
            pl.BlockSpec((1, d_out), lambda i: (0, 0)),
        ],
        out_specs=[
            pl.BlockSpec((_BN, d_in), lambda i: (i, 0)),
            pl.BlockSpec((_BN, d_out), lambda i: (i, 0)),
        ],
        out_shape=[
            jax.ShapeDtypeStruct((N, d_in), jnp.float32),
            jax.ShapeDtypeStruct((N, d_out), jnp.float32),
        ],
    )(s_prev, ssum, deg, w_self, b.reshape(1, -1))


def _tc_last(s_prev, ssum, deg, w_neigh):
    d_in = w_neigh.shape[0]
    d_out = w_neigh.shape[1]

    def body(sp_ref, sum_ref, deg_ref, wn_ref, o_ref):
        total = sum_ref[0] + sum_ref[1]
        degc = deg_ref[0, :, 0:1] + deg_ref[1, :, 0:1]
        h_neigh = total / jnp.maximum(degc, 1.0)
        o_ref[...] = sp_ref[...] + jnp.dot(
            h_neigh, wn_ref[...], preferred_element_type=jnp.float32)

    return pl.pallas_call(
        body,
        grid=(N // _BN,),
        in_specs=[
            pl.BlockSpec((_BN, d_out), lambda i: (i, 0)),
            pl.BlockSpec((NC, _BN, d_in), lambda i: (0, i, 0)),
            pl.BlockSpec((NC, _BN, DEGW), lambda i: (0, i, 0)),
            pl.BlockSpec(w_neigh.shape, lambda i: (0, 0)),
        ],
        out_specs=pl.BlockSpec((_BN, d_out), lambda i: (i, 0)),
        out_shape=jax.ShapeDtypeStruct((N, d_out), jnp.float32),
    )(s_prev, ssum, deg, w_neigh)


def kernel(inputs, edge_index0, edge_index1, edge_index2,
           W_self0, W_neigh0, b0, W_self1, W_neigh1, b1,
           W_self2, W_neigh2, b2):
    g0, s0 = _tc_first(inputs, W_neigh0, W_self0, b0)
    sum0 = _sc_sum(g0, edge_index0[0], edge_index0[1])
    deg0 = _sc_deg(edge_index0[1])
    g1, s1 = _tc_mid(s0, sum0, deg0, W_neigh1, W_self1, b1)
    sum1 = _sc_sum(g1, edge_index1[0], edge_index1[1])
    deg1 = _sc_deg(edge_index1[1])
    h2, s2 = _tc_mid2(s1, sum1, deg1, W_self2, b2)
    sum2 = _sc_sum(h2, edge_index2[0], edge_index2[1])
    deg2 = _sc_deg(edge_index2[1])
    return _tc_last(s2, sum2, deg2, W_neigh2)
